# baseline (speedup 1.0000x reference)
import jax
import jax.numpy as jnp
from jax.experimental import pallas as pl
from jax.experimental.pallas import tpu as pltpu


def _body(x_ref, r_ref, c_ref, o_ref):
    x = x_ref[...]
    r = r_ref[...]
    c = c_ref[...]
    half = r.shape[-1]
    o_ref[:, :, :, :half] = x[:, :, :, :half] + r[None, :, None, :]
    o_ref[:, :, :, half:] = x[:, :, :, half:] + c[None, None, :, :]


def kernel(inputs, row_emb, col_emb):
    B, R, Cg, C = inputs.shape
    BB, RB = 16, 8
    out = pl.pallas_call(
        _body,
        grid=(B // BB, R // RB),
        in_specs=[
            pl.BlockSpec((BB, RB, Cg, C), lambda b, rr: (b, rr, 0, 0)),
            pl.BlockSpec((RB, C // 2), lambda b, rr: (rr, 0)),
            pl.BlockSpec((Cg, C // 2), lambda b, rr: (0, 0)),
        ],
        out_specs=pl.BlockSpec((BB, RB, Cg, C), lambda b, rr: (b, rr, 0, 0)),
        out_shape=jax.ShapeDtypeStruct((B, R, Cg, C), inputs.dtype),
    )(inputs, row_emb, col_emb)
    return out.reshape(B, R * Cg, C)
